# baseline (speedup 1.0000x reference)
import jax
import jax.numpy as jnp
from jax import lax
from jax.experimental import pallas as pl
from jax.experimental.pallas import tpu as pltpu
from jax.experimental.pallas import tpu_sc as plsc

N = 10000
D = 128
OUT = 64

NC = 2
NS = 16
NW = NC * NS
L = 16

CHUNK = 128
NBUF = 3
SBUF = 3
DBUF = 4
UNROLL = 12
CPT = 80
EDGE_SLOTS = NW * CPT * CHUNK
ACC_ROWS = 10112
ZROWS = ACC_ROWS // NS
DEG_ROWS = 10240
DEG_ZROWS = DEG_ROWS // NS

TC_BLK = 2000


def _mesh():
    return plsc.VectorSubcoreMesh(core_axis_name="c", subcore_axis_name="s")


XROWS = 320
XLAST = N - 31 * XROWS


def _rsqrt_fast(v):
    y = lax.bitcast_convert_type(
        jnp.int32(0x5F3759DF) - (lax.bitcast_convert_type(v, jnp.int32) >> 1),
        jnp.float32)
    for _ in range(3):
        y = y * (1.5 - 0.5 * v * y * y)
    return y


def _deg_body(dst_hbm, x_hbm, deg_hbm, g0_hbm, dacc, dstv, xv, cntv,
              ones_v, zb, dsem):
    c = lax.axis_index("c")
    s = lax.axis_index("s")
    wid = c * NS + s
    zeros = jnp.zeros((L,), jnp.float32)
    ones = jnp.ones((L,), jnp.float32)
    for k in range(DEG_ZROWS // L):
        zb[pl.ds(k * L, L)] = zeros
    for k in range(CHUNK // L):
        ones_v[pl.ds(k * L, L)] = ones
    base = pl.multiple_of(s * DEG_ZROWS, 128)
    pltpu.sync_copy(zb, dacc.at[pl.ds(base, DEG_ZROWS)])
    plsc.subcore_barrier()

    for w in (s, s + NS):
        pltpu.sync_copy(dst_hbm.at[w], dstv)

        def step(j, carry):
            pltpu.async_copy(ones_v, dacc.at[dstv.at[j]], dsem, add=True)
            return carry

        lax.fori_loop(0, CPT, step, 0)

        def drain(j, carry):
            pltpu.make_async_copy(ones_v, dacc.at[dstv.at[j]], dsem).wait()
            return carry

        lax.fori_loop(0, CPT, drain, 0)
    plsc.subcore_barrier()

    @pl.when(c == 0)
    def _():
        pltpu.sync_copy(dacc.at[pl.ds(base, DEG_ZROWS)],
                        deg_hbm.at[pl.ds(base, DEG_ZROWS)])

    row_base = wid * XROWS
    off = (wid % 2) * 64
    wbase = pl.multiple_of(row_base - off, 128)
    pltpu.sync_copy(dacc.at[pl.ds(wbase, 384)], cntv)
    for i in range(384 // L):
        v = cntv[pl.ds(i * L, L)] + 1.0
        cntv[pl.ds(i * L, L)] = _rsqrt_fast(v)

    def scale_rows(nrows):
        pltpu.sync_copy(x_hbm.at[pl.ds(row_base, nrows), :],
                        xv.at[pl.ds(0, nrows), :])

        def sgroup(g, carry):
            cv = cntv[pl.ds(off + g * L, L)]
            dnums = lax.GatherDimensionNumbers(
                offset_dims=(), collapsed_slice_dims=(0,),
                start_index_map=(0,))
            for l in range(L):
                r = g * L + l
                db = lax.gather(cv, jnp.full((L, 1), l, jnp.int32), dnums,
                                (1,),
                                mode=lax.GatherScatterMode.PROMISE_IN_BOUNDS)
                for k in range(D // L):
                    xv[r, pl.ds(k * L, L)] = xv[r, pl.ds(k * L, L)] * db
            return carry

        lax.fori_loop(0, nrows // L, sgroup, 0)
        pltpu.sync_copy(xv.at[pl.ds(0, nrows), :],
                        g0_hbm.at[pl.ds(row_base, nrows), :])

    @pl.when(wid < 31)
    def _():
        scale_rows(XROWS)

    @pl.when(wid == 31)
    def _():
        scale_rows(XLAST)


_deg_kernel = pl.kernel(
    _deg_body,
    out_type=(jax.ShapeDtypeStruct((DEG_ROWS,), jnp.float32),
              jax.ShapeDtypeStruct((N, D), jnp.float32)),
    mesh=_mesh(),
    scratch_types=[
        pltpu.VMEM_SHARED((DEG_ROWS,), jnp.float32),
        pltpu.VMEM((CPT, CHUNK), jnp.int32),
        pltpu.VMEM((XROWS, D), jnp.float32),
        pltpu.VMEM((384,), jnp.float32),
        pltpu.VMEM((CHUNK,), jnp.float32),
        pltpu.VMEM((DEG_ZROWS,), jnp.float32),
        pltpu.SemaphoreType.DMA,
    ],
)


def _hop_body(g_hbm, src_hbm, dst_hbm, out_hbm, acc, sidx, didx, rows,
              gsem, ssem, issem, idsem):
    c = lax.axis_index("c")
    s = lax.axis_index("s")
    wid = c * NS + s
    zeros = jnp.zeros((L,), jnp.float32)

    def zrow(r, carry):
        for k in range(D // L):
            rows[0, r, pl.ds(k * L, L)] = zeros
        return carry

    lax.fori_loop(0, CHUNK, zrow, 0)
    base = pl.multiple_of(s * ZROWS, 8)
    for k in range(ZROWS // CHUNK):
        pltpu.async_copy(rows.at[0],
                         acc.at[pl.ds(base + k * CHUNK, CHUNK), :],
                         ssem.at[0])
    pltpu.async_copy(rows.at[0].at[pl.ds(0, ZROWS % CHUNK)],
                     acc.at[pl.ds(base + ZROWS - ZROWS % CHUNK,
                                  ZROWS % CHUNK), :],
                     ssem.at[1])
    for jj in range(3):
        pltpu.async_copy(src_hbm.at[wid].at[jj], sidx.at[jj], issem.at[jj])
        pltpu.async_copy(dst_hbm.at[wid].at[jj], didx.at[jj], idsem.at[jj])

    def wait_sidx(j, us):
        pltpu.make_async_copy(src_hbm.at[wid].at[j], sidx.at[us],
                              issem.at[us]).wait()

    def wait_didx(j, ud):
        pltpu.make_async_copy(dst_hbm.at[wid].at[j], didx.at[ud],
                              idsem.at[ud]).wait()

    def start_gather(us, b3):
        pltpu.async_copy(g_hbm.at[sidx.at[us]], rows.at[b3], gsem.at[b3])

    def wait_gather(us, b3):
        pltpu.make_async_copy(g_hbm.at[sidx.at[us]], rows.at[b3],
                              gsem.at[b3]).wait()

    def start_scatter(ud, b3):
        pltpu.async_copy(rows.at[b3], acc.at[didx.at[ud]], ssem.at[b3],
                         add=True)

    def wait_scatter(ud, b3):
        pltpu.make_async_copy(rows.at[b3], acc.at[didx.at[ud]],
                              ssem.at[b3]).wait()

    wait_sidx(1, 1)
    start_gather(1, 1)
    for k in range(ZROWS // CHUNK):
        pltpu.make_async_copy(rows.at[0],
                              acc.at[pl.ds(base + k * CHUNK, CHUNK), :],
                              ssem.at[0]).wait()
    pltpu.make_async_copy(rows.at[0].at[pl.ds(0, ZROWS % CHUNK)],
                          acc.at[pl.ds(base + ZROWS - ZROWS % CHUNK,
                                       ZROWS % CHUNK), :],
                          ssem.at[1]).wait()
    wait_sidx(0, 0)
    start_gather(0, 0)
    plsc.subcore_barrier()

    def emit_step(j, u):
        b3 = u % NBUF
        us = u % SBUF
        ud = u % DBUF
        wait_gather(us, b3)
        wait_didx(j, ud)
        start_scatter(ud, b3)

        @pl.when(j >= 1)
        def _():
            wait_scatter((u - 1) % DBUF, (u - 1) % NBUF)

        nl = j + 3

        @pl.when(nl < CPT)
        def _():
            pltpu.async_copy(src_hbm.at[wid].at[nl], sidx.at[(u + 3) % SBUF],
                             issem.at[(u + 3) % SBUF])
            pltpu.async_copy(dst_hbm.at[wid].at[nl], didx.at[(u + 3) % DBUF],
                             idsem.at[(u + 3) % DBUF])

        ng = j + 2

        @pl.when(ng < CPT)
        def _():
            wait_sidx(ng, (u + 2) % SBUF)
            start_gather((u + 2) % SBUF, (u + 2) % NBUF)

    def step(t, carry):
        for u in range(UNROLL):
            emit_step(t * UNROLL + u, u)
        return carry

    lax.fori_loop(0, CPT // UNROLL, step, 0)
    for j in range(CPT - CPT % UNROLL, CPT):
        emit_step(j, j % UNROLL)
    wait_scatter((CPT - 1) % DBUF, (CPT - 1) % NBUF)
    plsc.subcore_barrier()
    pltpu.sync_copy(acc.at[pl.ds(base, ZROWS), :],
                    out_hbm.at[c].at[pl.ds(base, ZROWS), :])


_hop_kernel = pl.kernel(
    _hop_body,
    out_type=jax.ShapeDtypeStruct((NC, ACC_ROWS, D), jnp.float32),
    mesh=_mesh(),
    scratch_types=[
        pltpu.VMEM_SHARED((ACC_ROWS, D), jnp.float32),
        pltpu.VMEM((SBUF, CHUNK), jnp.int32),
        pltpu.VMEM((DBUF, CHUNK), jnp.int32),
        pltpu.VMEM((NBUF, CHUNK, D), jnp.float32),
        pltpu.SemaphoreType.DMA((NBUF,)),
        pltpu.SemaphoreType.DMA((NBUF,)),
        pltpu.SemaphoreType.DMA((SBUF,)),
        pltpu.SemaphoreType.DMA((DBUF,)),
    ],
)


def _combine_mid_body(p0_ref, p1_ref, g_ref, dcol_ref, o_ref):
    o_ref[...] = (p0_ref[0] + p1_ref[0] + g_ref[...]) / dcol_ref[...]


def _final_body(p0_ref, p1_ref, g_ref, dcol_ref, wc_ref, bc_ref, wh_ref,
                bh_ref, o_ref):
    h = (p0_ref[0] + p1_ref[0] + g_ref[...]) * lax.rsqrt(dcol_ref[...])
    t = jnp.dot(h, wc_ref[...], preferred_element_type=jnp.float32)
    t = t + bc_ref[...]
    o = jnp.dot(t, wh_ref[...], preferred_element_type=jnp.float32)
    o_ref[...] = o + bh_ref[...]


_row_spec = pl.BlockSpec((TC_BLK, D), lambda i: (i, 0))
_col_spec = pl.BlockSpec((TC_BLK, 1), lambda i: (i, 0))
_p0_spec = pl.BlockSpec((1, TC_BLK, D), lambda i: (0, i, 0))
_p1_spec = pl.BlockSpec((1, TC_BLK, D), lambda i: (1, i, 0))

_combine_mid = pl.pallas_call(
    _combine_mid_body,
    grid=(N // TC_BLK,),
    in_specs=[_p0_spec, _p1_spec, _row_spec, _col_spec],
    out_specs=_row_spec,
    out_shape=jax.ShapeDtypeStruct((N, D), jnp.float32),
)

_final = pl.pallas_call(
    _final_body,
    grid=(N // TC_BLK,),
    in_specs=[
        _p0_spec, _p1_spec, _row_spec, _col_spec,
        pl.BlockSpec((D, D), lambda i: (0, 0)),
        pl.BlockSpec((1, D), lambda i: (0, 0)),
        pl.BlockSpec((D, OUT), lambda i: (0, 0)),
        pl.BlockSpec((1, OUT), lambda i: (0, 0)),
    ],
    out_specs=pl.BlockSpec((TC_BLK, OUT), lambda i: (i, 0)),
    out_shape=jax.ShapeDtypeStruct((N, OUT), jnp.float32),
)


@jax.jit
def kernel(x, edge_index, W_conv, b_conv, W_head, b_head):
    pad = EDGE_SLOTS - edge_index.shape[1]
    ar = jnp.arange(pad, dtype=jnp.int32)
    src = jnp.concatenate([edge_index[0], (ar * 13) % N])
    dst = jnp.concatenate([edge_index[1], N + ar % (ACC_ROWS - N)])
    src = src.reshape(NW, CPT, CHUNK)
    dst = dst.reshape(NW, CPT, CHUNK)

    deg, g0 = _deg_kernel(dst, x)
    dcol = (deg[:N] + 1.0).reshape(N, 1)

    p = _hop_kernel(g0, src, dst)
    g1 = _combine_mid(p, p, g0, dcol)
    p = _hop_kernel(g1, src, dst)
    return _final(p, p, g1, dcol, W_conv, b_conv.reshape(1, D),
                  W_head, b_head.reshape(1, OUT))


# speedup vs baseline: 1.0084x; 1.0084x over previous
import jax
import jax.numpy as jnp
from jax import lax
from jax.experimental import pallas as pl
from jax.experimental.pallas import tpu as pltpu
from jax.experimental.pallas import tpu_sc as plsc

N = 10000
D = 128
OUT = 64

NC = 2
NS = 16
NW = NC * NS
L = 16

CHUNK = 128
NBUF = 3
SBUF = 3
DBUF = 4
UNROLL = 12
CPT = 80
EDGE_SLOTS = NW * CPT * CHUNK
ACC_ROWS = 10112
ZROWS = ACC_ROWS // NS
DEG_ROWS = 10240
DEG_ZROWS = DEG_ROWS // NS

TC_BLK = 2000


def _mesh():
    return plsc.VectorSubcoreMesh(core_axis_name="c", subcore_axis_name="s")


XROWS = 320
XLAST = N - 31 * XROWS


def _rsqrt_fast(v):
    y = lax.bitcast_convert_type(
        jnp.int32(0x5F3759DF) - (lax.bitcast_convert_type(v, jnp.int32) >> 1),
        jnp.float32)
    for _ in range(3):
        y = y * (1.5 - 0.5 * v * y * y)
    return y


def _deg_body(dst_hbm, x_hbm, deg_hbm, g0_hbm, dacc, dstv, xv, cntv,
              ones_v, zb, dsem, xsem):
    c = lax.axis_index("c")
    s = lax.axis_index("s")
    wid = c * NS + s
    row_base = wid * XROWS

    @pl.when(wid < 31)
    def _():
        pltpu.async_copy(x_hbm.at[pl.ds(row_base, XROWS), :],
                         xv.at[pl.ds(0, XROWS), :], xsem)

    @pl.when(wid == 31)
    def _():
        pltpu.async_copy(x_hbm.at[pl.ds(row_base, XLAST), :],
                         xv.at[pl.ds(0, XLAST), :], xsem)

    zeros = jnp.zeros((L,), jnp.float32)
    ones = jnp.ones((L,), jnp.float32)
    for k in range(DEG_ZROWS // L):
        zb[pl.ds(k * L, L)] = zeros
    for k in range(CHUNK // L):
        ones_v[pl.ds(k * L, L)] = ones
    base = pl.multiple_of(s * DEG_ZROWS, 128)
    pltpu.sync_copy(zb, dacc.at[pl.ds(base, DEG_ZROWS)])
    plsc.subcore_barrier()

    for w in (s, s + NS):
        pltpu.sync_copy(dst_hbm.at[w], dstv)

        def step(j, carry):
            pltpu.async_copy(ones_v, dacc.at[dstv.at[j]], dsem, add=True)
            return carry

        lax.fori_loop(0, CPT, step, 0)

        def drain(j, carry):
            pltpu.make_async_copy(ones_v, dacc.at[dstv.at[j]], dsem).wait()
            return carry

        lax.fori_loop(0, CPT, drain, 0)
    plsc.subcore_barrier()

    @pl.when(c == 0)
    def _():
        pltpu.async_copy(dacc.at[pl.ds(base, DEG_ZROWS)],
                         deg_hbm.at[pl.ds(base, DEG_ZROWS)], dsem)

    off = (wid % 2) * 64
    wbase = pl.multiple_of(row_base - off, 128)
    pltpu.sync_copy(dacc.at[pl.ds(wbase, 384)], cntv)
    for i in range(384 // L):
        v = cntv[pl.ds(i * L, L)] + 1.0
        cntv[pl.ds(i * L, L)] = _rsqrt_fast(v)

    def scale_rows(nrows):
        pltpu.make_async_copy(x_hbm.at[pl.ds(row_base, nrows), :],
                              xv.at[pl.ds(0, nrows), :], xsem).wait()

        def sgroup(g, carry):
            cv = cntv[pl.ds(off + g * L, L)]
            dnums = lax.GatherDimensionNumbers(
                offset_dims=(), collapsed_slice_dims=(0,),
                start_index_map=(0,))
            for l in range(L):
                r = g * L + l
                db = lax.gather(cv, jnp.full((L, 1), l, jnp.int32), dnums,
                                (1,),
                                mode=lax.GatherScatterMode.PROMISE_IN_BOUNDS)
                for k in range(D // L):
                    xv[r, pl.ds(k * L, L)] = xv[r, pl.ds(k * L, L)] * db
            return carry

        lax.fori_loop(0, nrows // L, sgroup, 0)
        pltpu.sync_copy(xv.at[pl.ds(0, nrows), :],
                        g0_hbm.at[pl.ds(row_base, nrows), :])

    @pl.when(wid < 31)
    def _():
        scale_rows(XROWS)

    @pl.when(wid == 31)
    def _():
        scale_rows(XLAST)

    @pl.when(c == 0)
    def _():
        pltpu.make_async_copy(dacc.at[pl.ds(base, DEG_ZROWS)],
                              deg_hbm.at[pl.ds(base, DEG_ZROWS)],
                              dsem).wait()


_deg_kernel = pl.kernel(
    _deg_body,
    out_type=(jax.ShapeDtypeStruct((DEG_ROWS,), jnp.float32),
              jax.ShapeDtypeStruct((N, D), jnp.float32)),
    mesh=_mesh(),
    scratch_types=[
        pltpu.VMEM_SHARED((DEG_ROWS,), jnp.float32),
        pltpu.VMEM((CPT, CHUNK), jnp.int32),
        pltpu.VMEM((XROWS, D), jnp.float32),
        pltpu.VMEM((384,), jnp.float32),
        pltpu.VMEM((CHUNK,), jnp.float32),
        pltpu.VMEM((DEG_ZROWS,), jnp.float32),
        pltpu.SemaphoreType.DMA,
        pltpu.SemaphoreType.DMA,
    ],
)


def _hop_body(g_hbm, src_hbm, dst_hbm, out_hbm, acc, sidx, didx, rows,
              gsem, ssem, issem, idsem):
    c = lax.axis_index("c")
    s = lax.axis_index("s")
    wid = c * NS + s
    zeros = jnp.zeros((L,), jnp.float32)

    def zrow(r, carry):
        for k in range(D // L):
            rows[0, r, pl.ds(k * L, L)] = zeros
        return carry

    lax.fori_loop(0, CHUNK, zrow, 0)
    base = pl.multiple_of(s * ZROWS, 8)
    for k in range(ZROWS // CHUNK):
        pltpu.async_copy(rows.at[0],
                         acc.at[pl.ds(base + k * CHUNK, CHUNK), :],
                         ssem.at[0])
    pltpu.async_copy(rows.at[0].at[pl.ds(0, ZROWS % CHUNK)],
                     acc.at[pl.ds(base + ZROWS - ZROWS % CHUNK,
                                  ZROWS % CHUNK), :],
                     ssem.at[1])
    for jj in range(3):
        pltpu.async_copy(src_hbm.at[wid].at[jj], sidx.at[jj], issem.at[jj])
        pltpu.async_copy(dst_hbm.at[wid].at[jj], didx.at[jj], idsem.at[jj])

    def wait_sidx(j, us):
        pltpu.make_async_copy(src_hbm.at[wid].at[j], sidx.at[us],
                              issem.at[us]).wait()

    def wait_didx(j, ud):
        pltpu.make_async_copy(dst_hbm.at[wid].at[j], didx.at[ud],
                              idsem.at[ud]).wait()

    def start_gather(us, b3):
        pltpu.async_copy(g_hbm.at[sidx.at[us]], rows.at[b3], gsem.at[b3])

    def wait_gather(us, b3):
        pltpu.make_async_copy(g_hbm.at[sidx.at[us]], rows.at[b3],
                              gsem.at[b3]).wait()

    def start_scatter(ud, b3):
        pltpu.async_copy(rows.at[b3], acc.at[didx.at[ud]], ssem.at[b3],
                         add=True)

    def wait_scatter(ud, b3):
        pltpu.make_async_copy(rows.at[b3], acc.at[didx.at[ud]],
                              ssem.at[b3]).wait()

    wait_sidx(1, 1)
    start_gather(1, 1)
    for k in range(ZROWS // CHUNK):
        pltpu.make_async_copy(rows.at[0],
                              acc.at[pl.ds(base + k * CHUNK, CHUNK), :],
                              ssem.at[0]).wait()
    pltpu.make_async_copy(rows.at[0].at[pl.ds(0, ZROWS % CHUNK)],
                          acc.at[pl.ds(base + ZROWS - ZROWS % CHUNK,
                                       ZROWS % CHUNK), :],
                          ssem.at[1]).wait()
    wait_sidx(0, 0)
    start_gather(0, 0)
    plsc.subcore_barrier()

    def emit_step(j, u):
        b3 = u % NBUF
        us = u % SBUF
        ud = u % DBUF
        wait_gather(us, b3)
        wait_didx(j, ud)
        start_scatter(ud, b3)

        @pl.when(j >= 1)
        def _():
            wait_scatter((u - 1) % DBUF, (u - 1) % NBUF)

        nl = j + 3

        @pl.when(nl < CPT)
        def _():
            pltpu.async_copy(src_hbm.at[wid].at[nl], sidx.at[(u + 3) % SBUF],
                             issem.at[(u + 3) % SBUF])
            pltpu.async_copy(dst_hbm.at[wid].at[nl], didx.at[(u + 3) % DBUF],
                             idsem.at[(u + 3) % DBUF])

        ng = j + 2

        @pl.when(ng < CPT)
        def _():
            wait_sidx(ng, (u + 2) % SBUF)
            start_gather((u + 2) % SBUF, (u + 2) % NBUF)

    def step(t, carry):
        for u in range(UNROLL):
            emit_step(t * UNROLL + u, u)
        return carry

    lax.fori_loop(0, CPT // UNROLL, step, 0)
    for j in range(CPT - CPT % UNROLL, CPT):
        emit_step(j, j % UNROLL)
    wait_scatter((CPT - 1) % DBUF, (CPT - 1) % NBUF)
    plsc.subcore_barrier()
    pltpu.sync_copy(acc.at[pl.ds(base, ZROWS), :],
                    out_hbm.at[c].at[pl.ds(base, ZROWS), :])


_hop_kernel = pl.kernel(
    _hop_body,
    out_type=jax.ShapeDtypeStruct((NC, ACC_ROWS, D), jnp.float32),
    mesh=_mesh(),
    scratch_types=[
        pltpu.VMEM_SHARED((ACC_ROWS, D), jnp.float32),
        pltpu.VMEM((SBUF, CHUNK), jnp.int32),
        pltpu.VMEM((DBUF, CHUNK), jnp.int32),
        pltpu.VMEM((NBUF, CHUNK, D), jnp.float32),
        pltpu.SemaphoreType.DMA((NBUF,)),
        pltpu.SemaphoreType.DMA((NBUF,)),
        pltpu.SemaphoreType.DMA((SBUF,)),
        pltpu.SemaphoreType.DMA((DBUF,)),
    ],
)


def _combine_mid_body(p0_ref, p1_ref, g_ref, dcol_ref, o_ref):
    o_ref[...] = (p0_ref[0] + p1_ref[0] + g_ref[...]) / dcol_ref[...]


def _final_body(p0_ref, p1_ref, g_ref, dcol_ref, wc_ref, bc_ref, wh_ref,
                bh_ref, o_ref):
    h = (p0_ref[0] + p1_ref[0] + g_ref[...]) * lax.rsqrt(dcol_ref[...])
    t = jnp.dot(h, wc_ref[...], preferred_element_type=jnp.float32)
    t = t + bc_ref[...]
    o = jnp.dot(t, wh_ref[...], preferred_element_type=jnp.float32)
    o_ref[...] = o + bh_ref[...]


_row_spec = pl.BlockSpec((TC_BLK, D), lambda i: (i, 0))
_col_spec = pl.BlockSpec((TC_BLK, 1), lambda i: (i, 0))
_p0_spec = pl.BlockSpec((1, TC_BLK, D), lambda i: (0, i, 0))
_p1_spec = pl.BlockSpec((1, TC_BLK, D), lambda i: (1, i, 0))

_combine_mid = pl.pallas_call(
    _combine_mid_body,
    grid=(N // TC_BLK,),
    in_specs=[_p0_spec, _p1_spec, _row_spec, _col_spec],
    out_specs=_row_spec,
    out_shape=jax.ShapeDtypeStruct((N, D), jnp.float32),
)

_final = pl.pallas_call(
    _final_body,
    grid=(N // TC_BLK,),
    in_specs=[
        _p0_spec, _p1_spec, _row_spec, _col_spec,
        pl.BlockSpec((D, D), lambda i: (0, 0)),
        pl.BlockSpec((1, D), lambda i: (0, 0)),
        pl.BlockSpec((D, OUT), lambda i: (0, 0)),
        pl.BlockSpec((1, OUT), lambda i: (0, 0)),
    ],
    out_specs=pl.BlockSpec((TC_BLK, OUT), lambda i: (i, 0)),
    out_shape=jax.ShapeDtypeStruct((N, OUT), jnp.float32),
)


@jax.jit
def kernel(x, edge_index, W_conv, b_conv, W_head, b_head):
    pad = EDGE_SLOTS - edge_index.shape[1]
    ar = jnp.arange(pad, dtype=jnp.int32)
    src = jnp.concatenate([edge_index[0], (ar * 13) % N])
    dst = jnp.concatenate([edge_index[1], N + ar % (ACC_ROWS - N)])
    src = src.reshape(NW, CPT, CHUNK)
    dst = dst.reshape(NW, CPT, CHUNK)

    deg, g0 = _deg_kernel(dst, x)
    dcol = (deg[:N] + 1.0).reshape(N, 1)

    p = _hop_kernel(g0, src, dst)
    g1 = _combine_mid(p, p, g0, dcol)
    p = _hop_kernel(g1, src, dst)
    return _final(p, p, g1, dcol, W_conv, b_conv.reshape(1, D),
                  W_head, b_head.reshape(1, OUT))
